# baseline (speedup 1.0000x reference)
import functools
import math

import jax
import jax.numpy as jnp
from jax import lax
from jax.experimental import pallas as pl
from jax.experimental.pallas import tpu as pltpu
from jax.experimental.pallas import tpu_sc as plsc

T, D, H, E, TOPK = 2048, 1024, 4096, 8, 2
KH = 512
NKH = H // KH
TM = 256
NP = 4096 + E * TM
NW = 32
TW = T // NW


def _router_body(x_ref, wg_ref, bg_ref, logits_ref, idx_ref, w0_ref, w1_ref,
                 pos0_ref, pos1_ref, poffs_ref, poffe_ref):
    x = x_ref[...]
    logits = lax.dot_general(
        x, wg_ref[...], (((1,), (0,)), ((), ())),
        preferred_element_type=jnp.float32,
    ) + bg_ref[...][None, :]
    logits_ref[...] = logits

    iota = lax.broadcasted_iota(jnp.int32, (T, E), 1)
    max1 = jnp.max(logits, axis=1, keepdims=True)
    idx1 = jnp.min(jnp.where(logits == max1, iota, E), axis=1, keepdims=True)
    masked = jnp.where(iota == idx1, -jnp.inf, logits)
    max2 = jnp.max(masked, axis=1, keepdims=True)
    idx2 = jnp.min(jnp.where(masked == max2, iota, E), axis=1, keepdims=True)
    idx_ref[...] = jnp.concatenate([idx1, idx2], axis=1)

    w1 = 1.0 / (1.0 + jnp.exp(max2 - max1))
    w0_ref[...] = w1
    w1_ref[...] = 1.0 - w1

    oh1 = (iota == idx1).astype(jnp.float32)
    oh2 = (iota == idx2).astype(jnp.float32)
    m = oh1 + oh2
    tril = (lax.broadcasted_iota(jnp.int32, (T, T), 0)
            >= lax.broadcasted_iota(jnp.int32, (T, T), 1)).astype(jnp.float32)
    sincl = lax.dot_general(tril, m, (((1,), (0,)), ((), ())),
                            preferred_element_type=jnp.float32)
    sexcl = sincl - m
    counts = sincl[T - 1:T, :]
    cnt_pad = jnp.ceil(counts / TM) * TM
    triu = (lax.broadcasted_iota(jnp.int32, (E, E), 0)
            < lax.broadcasted_iota(jnp.int32, (E, E), 1)).astype(jnp.float32)
    poff = lax.dot_general(cnt_pad, triu, (((1,), (0,)), ((), ())),
                           preferred_element_type=jnp.float32)
    poffs_ref[...] = poff.astype(jnp.int32)
    poffe_ref[...] = (poff + cnt_pad).astype(jnp.int32)

    dest = poff + sexcl
    pos0_ref[...] = jnp.sum(oh1 * dest, axis=1, keepdims=True).astype(jnp.int32)
    pos1_ref[...] = jnp.sum(oh2 * dest, axis=1, keepdims=True).astype(jnp.int32)


_router = pl.pallas_call(
    _router_body,
    out_shape=(
        jax.ShapeDtypeStruct((T, E), jnp.float32),
        jax.ShapeDtypeStruct((T, TOPK), jnp.int32),
        jax.ShapeDtypeStruct((T, 1), jnp.float32),
        jax.ShapeDtypeStruct((T, 1), jnp.float32),
        jax.ShapeDtypeStruct((T, 1), jnp.int32),
        jax.ShapeDtypeStruct((T, 1), jnp.int32),
        jax.ShapeDtypeStruct((1, E), jnp.int32),
        jax.ShapeDtypeStruct((1, E), jnp.int32),
    ),
)


@functools.lru_cache(maxsize=None)
def _get_sc_kernels():
    mesh = plsc.VectorSubcoreMesh(core_axis_name="c", subcore_axis_name="s")

    @functools.partial(
        pl.kernel,
        mesh=mesh,
        out_type=jax.ShapeDtypeStruct((NP, D), jnp.float32),
        scratch_types=[
            pltpu.VMEM((TW, D), jnp.float32),
            pltpu.VMEM((TW,), jnp.int32),
            pltpu.VMEM((TW,), jnp.int32),
            pltpu.SemaphoreType.DMA,
            pltpu.SemaphoreType.DMA,
        ],
    )
    def _sc_dispatch(x_hbm, pos0_hbm, pos1_hbm, xs_hbm, xrows, p0v, p1v,
                     s0, s1):
        wid = lax.axis_index("s") * 2 + lax.axis_index("c")
        base = wid * TW
        pltpu.sync_copy(pos0_hbm.at[pl.ds(base, TW)], p0v)
        pltpu.sync_copy(pos1_hbm.at[pl.ds(base, TW)], p1v)
        pltpu.sync_copy(x_hbm.at[pl.ds(base, TW)], xrows)
        c0 = pltpu.async_copy(xrows, xs_hbm.at[p0v], s0)
        c1 = pltpu.async_copy(xrows, xs_hbm.at[p1v], s1)
        c0.wait()
        c1.wait()

    @functools.partial(
        pl.kernel,
        mesh=mesh,
        out_type=(
            jax.ShapeDtypeStruct((T, D), jnp.float32),
            jax.ShapeDtypeStruct((T, D), jnp.float32),
        ),
        scratch_types=[
            pltpu.VMEM((TW, D), jnp.float32),
            pltpu.VMEM((TW,), jnp.int32),
            pltpu.SemaphoreType.DMA,
        ],
    )
    def _sc_gather(ys_hbm, pos0_hbm, pos1_hbm, a_hbm, b_hbm, rows, pv, sem):
        wid = lax.axis_index("s") * 2 + lax.axis_index("c")
        base = wid * TW
        pltpu.sync_copy(pos0_hbm.at[pl.ds(base, TW)], pv)
        pltpu.async_copy(ys_hbm.at[pv], rows, sem).wait()
        pltpu.sync_copy(rows, a_hbm.at[pl.ds(base, TW)])
        pltpu.sync_copy(pos1_hbm.at[pl.ds(base, TW)], pv)
        pltpu.async_copy(ys_hbm.at[pv], rows, sem).wait()
        pltpu.sync_copy(rows, b_hbm.at[pl.ds(base, TW)])

    return _sc_dispatch, _sc_gather


def _tobf16_body(x_ref, o_ref):
    o_ref[...] = x_ref[...].astype(jnp.bfloat16)


_tobf16 = pl.pallas_call(
    _tobf16_body,
    grid=(4,),
    in_specs=[pl.BlockSpec((NP // 4, D), lambda i: (i, 0))],
    out_specs=pl.BlockSpec((NP // 4, D), lambda i: (i, 0)),
    out_shape=jax.ShapeDtypeStruct((NP, D), jnp.bfloat16),
)


def _gmm_body(poffs_ref, poffe_ref, xs_ref, w1_ref, b1_ref, w2_ref, b2_ref,
              ys_ref):
    e = pl.program_id(0)
    kh = pl.program_id(1)
    lo = poffs_ref[0, e]
    hi = poffe_ref[0, e]
    w1c = w1_ref[0].astype(jnp.bfloat16)
    w2c = w2_ref[0].astype(jnp.bfloat16)
    b1c = b1_ref[0, 0]
    b2c = b2_ref[0]

    def tile_body(mt, carry):
        r0 = mt * TM
        xt = xs_ref[pl.ds(r0, TM), :]
        h = jnp.dot(xt, w1c, preferred_element_type=jnp.float32) + b1c
        h = 0.5 * h * (1.0 + lax.erf(h * (1.0 / math.sqrt(2.0))))
        part = jnp.dot(h.astype(jnp.bfloat16), w2c,
                       preferred_element_type=jnp.float32)

        @pl.when(kh == 0)
        def _init():
            ys_ref[pl.ds(r0, TM), :] = part + b2c

        @pl.when(kh != 0)
        def _acc():
            ys_ref[pl.ds(r0, TM), :] += part

        return carry

    lax.fori_loop(lo // TM, hi // TM, tile_body, 0)


_gmm = pl.pallas_call(
    _gmm_body,
    grid=(E, NKH),
    in_specs=[
        pl.BlockSpec(memory_space=pltpu.SMEM),
        pl.BlockSpec(memory_space=pltpu.SMEM),
        pl.BlockSpec((NP, D), lambda e, kh: (0, 0)),
        pl.BlockSpec((1, D, KH), lambda e, kh: (e, 0, kh)),
        pl.BlockSpec((1, 1, 1, KH), lambda e, kh: (e, kh, 0, 0)),
        pl.BlockSpec((1, KH, D), lambda e, kh: (e, kh, 0)),
        pl.BlockSpec((1, 1, D), lambda e, kh: (e, 0, 0)),
    ],
    out_specs=pl.BlockSpec((NP, D), lambda e, kh: (0, 0)),
    out_shape=jax.ShapeDtypeStruct((NP, D), jnp.float32),
)


def _combine_body(a_ref, b_ref, w0_ref, w1_ref, out_ref):
    out_ref[...] = w0_ref[...] * a_ref[...] + w1_ref[...] * b_ref[...]


_combine = pl.pallas_call(
    _combine_body,
    out_shape=jax.ShapeDtypeStruct((T, D), jnp.float32),
)


@jax.jit
def kernel(x, Wg, bg, W1, b1, W2, b2):
    x_flat = x.reshape(T, D)

    (logits, idx, w0, w1, pos0, pos1, poffs, poffe) = _router(x_flat, Wg, bg)

    sc_dispatch, sc_gather = _get_sc_kernels()
    xs = sc_dispatch(x_flat, pos0.reshape(T), pos1.reshape(T))
    ys = _gmm(poffs, poffe, _tobf16(xs), W1, b1.reshape(E, NKH, 1, KH), W2,
              b2.reshape(E, 1, D))
    a, b = sc_gather(ys, pos0.reshape(T), pos1.reshape(T))
    out = _combine(a, b, w0, w1)

    return (out.reshape(1, T, D),
            logits.reshape(1, T, E),
            idx.reshape(1, T, TOPK))


# speedup vs baseline: 1.0144x; 1.0144x over previous
import functools
import math

import jax
import jax.numpy as jnp
from jax import lax
from jax.experimental import pallas as pl
from jax.experimental.pallas import tpu as pltpu
from jax.experimental.pallas import tpu_sc as plsc

T, D, H, E, TOPK = 2048, 1024, 4096, 8, 2
KH = 512
NKH = H // KH
TM = 256
NP = 4096 + E * TM
NW = 32
TW = T // NW


def _router_body(x_ref, wg_ref, bg_ref, logits_ref, idx_ref, w0_ref, w1_ref,
                 pos0_ref, pos1_ref, poffs_ref, poffe_ref):
    x = x_ref[...]
    logits = lax.dot_general(
        x, wg_ref[...], (((1,), (0,)), ((), ())),
        preferred_element_type=jnp.float32,
    ) + bg_ref[...][None, :]
    logits_ref[...] = logits

    iota = lax.broadcasted_iota(jnp.int32, (T, E), 1)
    max1 = jnp.max(logits, axis=1, keepdims=True)
    idx1 = jnp.min(jnp.where(logits == max1, iota, E), axis=1, keepdims=True)
    masked = jnp.where(iota == idx1, -jnp.inf, logits)
    max2 = jnp.max(masked, axis=1, keepdims=True)
    idx2 = jnp.min(jnp.where(masked == max2, iota, E), axis=1, keepdims=True)
    idx_ref[...] = jnp.concatenate([idx1, idx2], axis=1)

    w1 = 1.0 / (1.0 + jnp.exp(max2 - max1))
    w0_ref[...] = jnp.broadcast_to(w1, (T, 16))
    w1_ref[...] = jnp.broadcast_to(1.0 - w1, (T, 16))

    oh1 = (iota == idx1).astype(jnp.float32)
    oh2 = (iota == idx2).astype(jnp.float32)
    m = oh1 + oh2
    tril = (lax.broadcasted_iota(jnp.int32, (T, T), 0)
            >= lax.broadcasted_iota(jnp.int32, (T, T), 1)).astype(jnp.float32)
    sincl = lax.dot_general(tril, m, (((1,), (0,)), ((), ())),
                            preferred_element_type=jnp.float32)
    sexcl = sincl - m
    counts = sincl[T - 1:T, :]
    cnt_pad = jnp.ceil(counts / TM) * TM
    triu = (lax.broadcasted_iota(jnp.int32, (E, E), 0)
            < lax.broadcasted_iota(jnp.int32, (E, E), 1)).astype(jnp.float32)
    poff = lax.dot_general(cnt_pad, triu, (((1,), (0,)), ((), ())),
                           preferred_element_type=jnp.float32)
    poffs_ref[...] = poff.astype(jnp.int32)
    poffe_ref[...] = (poff + cnt_pad).astype(jnp.int32)

    dest = poff + sexcl
    pos0_ref[...] = jnp.sum(oh1 * dest, axis=1, keepdims=True).astype(jnp.int32)
    pos1_ref[...] = jnp.sum(oh2 * dest, axis=1, keepdims=True).astype(jnp.int32)


_router = pl.pallas_call(
    _router_body,
    out_shape=(
        jax.ShapeDtypeStruct((T, E), jnp.float32),
        jax.ShapeDtypeStruct((T, TOPK), jnp.int32),
        jax.ShapeDtypeStruct((T, 16), jnp.float32),
        jax.ShapeDtypeStruct((T, 16), jnp.float32),
        jax.ShapeDtypeStruct((T, 1), jnp.int32),
        jax.ShapeDtypeStruct((T, 1), jnp.int32),
        jax.ShapeDtypeStruct((1, E), jnp.int32),
        jax.ShapeDtypeStruct((1, E), jnp.int32),
    ),
)


@functools.lru_cache(maxsize=None)
def _get_sc_kernels():
    mesh = plsc.VectorSubcoreMesh(core_axis_name="c", subcore_axis_name="s")

    @functools.partial(
        pl.kernel,
        mesh=mesh,
        out_type=jax.ShapeDtypeStruct((NP, D), jnp.float32),
        scratch_types=[
            pltpu.VMEM((TW, D), jnp.float32),
            pltpu.VMEM((TW,), jnp.int32),
            pltpu.VMEM((TW,), jnp.int32),
            pltpu.SemaphoreType.DMA,
            pltpu.SemaphoreType.DMA,
        ],
    )
    def _sc_dispatch(x_hbm, pos0_hbm, pos1_hbm, xs_hbm, xrows, p0v, p1v,
                     s0, s1):
        wid = lax.axis_index("s") * 2 + lax.axis_index("c")
        base = wid * TW
        pltpu.sync_copy(pos0_hbm.at[pl.ds(base, TW)], p0v)
        pltpu.sync_copy(pos1_hbm.at[pl.ds(base, TW)], p1v)
        pltpu.sync_copy(x_hbm.at[pl.ds(base, TW)], xrows)
        c0 = pltpu.async_copy(xrows, xs_hbm.at[p0v], s0)
        c1 = pltpu.async_copy(xrows, xs_hbm.at[p1v], s1)
        c0.wait()
        c1.wait()

    CH = TW // 2

    @functools.partial(
        pl.kernel,
        mesh=mesh,
        out_type=jax.ShapeDtypeStruct((T, D), jnp.float32),
        scratch_types=[
            pltpu.VMEM((CH, D), jnp.float32),
            pltpu.VMEM((CH, D), jnp.float32),
            pltpu.VMEM((CH,), jnp.int32),
            pltpu.VMEM((CH,), jnp.int32),
            pltpu.VMEM((TW, 16), jnp.float32),
            pltpu.VMEM((TW, 16), jnp.float32),
            pltpu.SemaphoreType.DMA,
            pltpu.SemaphoreType.DMA,
        ],
    )
    def _sc_combine(ys_hbm, pos0_hbm, pos1_hbm, w0_hbm, w1_hbm, out_hbm,
                    arows, brows, p0v, p1v, w0v, w1v, s0, s1):
        wid = lax.axis_index("s") * 2 + lax.axis_index("c")
        base = wid * TW
        pltpu.sync_copy(w0_hbm.at[pl.ds(base, TW)], w0v)
        pltpu.sync_copy(w1_hbm.at[pl.ds(base, TW)], w1v)
        for c in range(TW // CH):
            co = c * CH
            pltpu.sync_copy(pos0_hbm.at[pl.ds(base + co, CH)], p0v)
            pltpu.sync_copy(pos1_hbm.at[pl.ds(base + co, CH)], p1v)
            c0 = pltpu.async_copy(ys_hbm.at[p0v], arows, s0)
            c1 = pltpu.async_copy(ys_hbm.at[p1v], brows, s1)
            c0.wait()
            c1.wait()

            def row_body(j, carry):
                w0b = w0v[co + j, :]
                w1b = w1v[co + j, :]
                for l in range(D // 16):
                    av = arows[j, pl.ds(l * 16, 16)]
                    bv = brows[j, pl.ds(l * 16, 16)]
                    arows[j, pl.ds(l * 16, 16)] = w0b * av + w1b * bv
                return carry

            lax.fori_loop(0, CH, row_body, 0)
            pltpu.sync_copy(arows, out_hbm.at[pl.ds(base + co, CH)])

    return _sc_dispatch, _sc_combine


def _tobf16_body(x_ref, o_ref):
    o_ref[...] = x_ref[...].astype(jnp.bfloat16)


_tobf16 = pl.pallas_call(
    _tobf16_body,
    grid=(4,),
    in_specs=[pl.BlockSpec((NP // 4, D), lambda i: (i, 0))],
    out_specs=pl.BlockSpec((NP // 4, D), lambda i: (i, 0)),
    out_shape=jax.ShapeDtypeStruct((NP, D), jnp.bfloat16),
)


def _gmm_body(poffs_ref, poffe_ref, xs_ref, w1_ref, b1_ref, w2_ref, b2_ref,
              ys_ref):
    e = pl.program_id(0)
    kh = pl.program_id(1)
    lo = poffs_ref[0, e]
    hi = poffe_ref[0, e]
    w1c = w1_ref[0]
    w2c = w2_ref[0]
    b1c = b1_ref[0, 0]
    b2c = b2_ref[0]

    def tile_body(mt, carry):
        r0 = mt * TM
        xt = xs_ref[pl.ds(r0, TM), :].astype(jnp.float32)
        h = jnp.dot(xt, w1c, preferred_element_type=jnp.float32) + b1c
        h = 0.5 * h * (1.0 + lax.erf(h * (1.0 / math.sqrt(2.0))))
        part = jnp.dot(h, w2c, preferred_element_type=jnp.float32)

        @pl.when(kh == 0)
        def _init():
            ys_ref[pl.ds(r0, TM), :] = part + b2c

        @pl.when(kh != 0)
        def _acc():
            ys_ref[pl.ds(r0, TM), :] += part

        return carry

    lax.fori_loop(lo // TM, hi // TM, tile_body, 0)


_gmm = pl.pallas_call(
    _gmm_body,
    grid=(E, NKH),
    in_specs=[
        pl.BlockSpec(memory_space=pltpu.SMEM),
        pl.BlockSpec(memory_space=pltpu.SMEM),
        pl.BlockSpec((NP, D), lambda e, kh: (0, 0)),
        pl.BlockSpec((1, D, KH), lambda e, kh: (e, 0, kh)),
        pl.BlockSpec((1, 1, 1, KH), lambda e, kh: (e, kh, 0, 0)),
        pl.BlockSpec((1, KH, D), lambda e, kh: (e, kh, 0)),
        pl.BlockSpec((1, 1, D), lambda e, kh: (e, 0, 0)),
    ],
    out_specs=pl.BlockSpec((NP, D), lambda e, kh: (0, 0)),
    out_shape=jax.ShapeDtypeStruct((NP, D), jnp.float32),
)


@jax.jit
def kernel(x, Wg, bg, W1, b1, W2, b2):
    x_flat = x.reshape(T, D)

    (logits, idx, w0, w1, pos0, pos1, poffs, poffe) = _router(x_flat, Wg, bg)

    sc_dispatch, sc_combine = _get_sc_kernels()
    xs = sc_dispatch(x_flat, pos0.reshape(T), pos1.reshape(T))
    ys = _gmm(poffs, poffe, _tobf16(xs), W1, b1.reshape(E, NKH, 1, KH), W2,
              b2.reshape(E, 1, D))
    out = sc_combine(ys, pos0.reshape(T), pos1.reshape(T), w0, w1)

    return (out.reshape(1, T, D),
            logits.reshape(1, T, E),
            idx.reshape(1, T, TOPK))
